# baseline (speedup 1.0000x reference)
import jax
import jax.numpy as jnp
from jax.experimental import pallas as pl

_NUM_BLOCKS = 64


def _body(idx_ref, x_ref, table_ref, out_ref):
    idx = idx_ref[0, 0, :]
    bb, s, d = x_ref.shape
    iota = jax.lax.broadcasted_iota(jnp.int32, (32, idx.shape[0]), 0)
    oht = (idx[None, :] == iota).astype(jnp.float32)
    emb = jax.lax.dot_general(
        oht, table_ref[...], (((0,), (0,)), ((), ())),
        preferred_element_type=jnp.float32,
    )
    out_ref[...] = x_ref[...] + emb.reshape(bb, s, d)


def kernel(inputs, x, table):
    B, S, D = x.shape
    BB = B // _NUM_BLOCKS
    R = BB * S
    idx3 = inputs.reshape(_NUM_BLOCKS, 1, R)
    return pl.pallas_call(
        _body,
        grid=(_NUM_BLOCKS,),
        in_specs=[
            pl.BlockSpec((1, 1, R), lambda i: (i, 0, 0)),
            pl.BlockSpec((BB, S, D), lambda i: (i, 0, 0)),
            pl.BlockSpec((32, D), lambda i: (0, 0)),
        ],
        out_specs=pl.BlockSpec((BB, S, D), lambda i: (i, 0, 0)),
        out_shape=jax.ShapeDtypeStruct((B, S, D), x.dtype),
    )(idx3, x, table)
